# baseline (speedup 1.0000x reference)
import numpy as np
import jax
import jax.numpy as jnp
from jax.experimental import pallas as pl
from jax.experimental.pallas import tpu as pltpu

DT = 0.01
MAX_T = 2.0
STEPS = 200
SP = 208
BN = 6144
BIGF = 1e9


def _ddm_block(x_ref, w_ref, a_ref, z_ref, ndt_ref, g_ref, noise_ref,
               pr_ref, pc_ref, xc_ref):
    a = a_ref[0, 0]
    z = z_ref[0, 0]
    ndt = ndt_ref[0, 0]
    gain = g_ref[0, 0]

    drift_dt = (gain * DT) * x_ref[...]
    th_hi = (a - 1e-6) - z * a
    th_lo = (-a + 1e-6) - z * a

    nz = noise_ref[...]
    hi = nz.astype(jnp.bfloat16)
    r1 = nz - hi.astype(jnp.float32)
    mid = r1.astype(jnp.bfloat16)
    lo = (r1 - mid.astype(jnp.float32)).astype(jnp.bfloat16)
    xc_ref[0:STEPS, :] = hi
    xc_ref[SP:SP + STEPS, :] = mid
    xc_ref[2 * SP:2 * SP + STEPS, :] = lo

    d_hi_f = drift_dt.astype(jnp.bfloat16).astype(jnp.float32)
    dr = drift_dt - d_hi_f
    d_mid_f = dr.astype(jnp.bfloat16).astype(jnp.float32)
    d_lo = dr - d_mid_f
    row8 = jax.lax.broadcasted_iota(jnp.int32, (8, BN), 0)
    m0 = row8 == 0
    xc_ref[STEPS:SP, :] = jnp.where(m0, drift_dt, 0.0).astype(jnp.bfloat16)
    xc_ref[SP + STEPS:2 * SP, :] = jnp.where(m0, dr, 0.0).astype(jnp.bfloat16)
    xc_ref[2 * SP + STEPS:3 * SP, :] = jnp.where(m0, d_lo, 0.0).astype(jnp.bfloat16)

    s = jax.lax.dot(w_ref[...], xc_ref[...],
                    preferred_element_type=jnp.float32)

    t_f = jax.lax.broadcasted_iota(
        jnp.int32, (STEPS, BN), 0).astype(jnp.float32)
    enc_u = jnp.where(s >= th_hi, t_f, BIGF)
    enc_l = jnp.where(s <= th_lo, t_f, BIGF)
    mu = jnp.min(enc_u, axis=0, keepdims=True)
    ml = jnp.min(enc_l, axis=0, keepdims=True)

    t_first = jnp.minimum(mu, ml)
    hit = t_first < BIGF
    pr_ref[...] = jnp.where(hit, t_first * DT + ndt, MAX_T + ndt)
    pc_ref[...] = jnp.where(hit, jnp.where(mu <= ml, 1.0, 0.0), 0.5)


def _weights():
    tril = np.tril(np.ones((STEPS, STEPS), np.float32))
    tcol = np.arange(1, STEPS + 1, dtype=np.float32).reshape(STEPS, 1)
    sec = np.concatenate([tril, tcol, np.zeros((STEPS, SP - STEPS - 1),
                                               np.float32)], axis=1)
    return jnp.asarray(np.concatenate([sec] * 3, axis=1), dtype=jnp.bfloat16)


@jax.jit
def kernel(x, a, z, ndt, drift_gain, noise):
    n = x.shape[0]
    x2 = x.reshape(1, n)
    w = _weights()
    grid = (pl.cdiv(n, BN),)
    scal = pl.BlockSpec(memory_space=pltpu.SMEM)
    pr, pc = pl.pallas_call(
        _ddm_block,
        grid=grid,
        in_specs=[
            pl.BlockSpec((1, BN), lambda i: (0, i)),
            pl.BlockSpec((STEPS, 3 * SP), lambda i: (0, 0)),
            scal, scal, scal, scal,
            pl.BlockSpec((STEPS, BN), lambda i: (0, i)),
        ],
        out_specs=[
            pl.BlockSpec((1, BN), lambda i: (0, i)),
            pl.BlockSpec((1, BN), lambda i: (0, i)),
        ],
        out_shape=[
            jax.ShapeDtypeStruct((1, n), jnp.float32),
            jax.ShapeDtypeStruct((1, n), jnp.float32),
        ],
        scratch_shapes=[pltpu.VMEM((3 * SP, BN), jnp.bfloat16)],
    )(x2, w,
      a.reshape(1, 1), z.reshape(1, 1), ndt.reshape(1, 1),
      drift_gain.reshape(1, 1), noise)
    return pr.reshape(n), pc.reshape(n)


# speedup vs baseline: 1.0018x; 1.0018x over previous
import numpy as np
import jax
import jax.numpy as jnp
from jax.experimental import pallas as pl
from jax.experimental.pallas import tpu as pltpu

DT = 0.01
MAX_T = 2.0
STEPS = 200
SP = 208
BN = 5120
BIGF = 1e9


def _ddm_block(x_ref, w_ref, a_ref, z_ref, ndt_ref, g_ref, noise_ref,
               pr_ref, pc_ref, xc_ref):
    a = a_ref[0, 0]
    z = z_ref[0, 0]
    ndt = ndt_ref[0, 0]
    gain = g_ref[0, 0]

    drift_dt = (gain * DT) * x_ref[...]
    th_hi = (a - 1e-6) - z * a
    th_lo = (-a + 1e-6) - z * a

    nz = noise_ref[...]
    hi = nz.astype(jnp.bfloat16)
    r1 = nz - hi.astype(jnp.float32)
    mid = r1.astype(jnp.bfloat16)
    lo = (r1 - mid.astype(jnp.float32)).astype(jnp.bfloat16)
    xc_ref[0:STEPS, :] = hi
    xc_ref[SP:SP + STEPS, :] = mid
    xc_ref[2 * SP:2 * SP + STEPS, :] = lo

    d_hi_f = drift_dt.astype(jnp.bfloat16).astype(jnp.float32)
    dr = drift_dt - d_hi_f
    d_mid_f = dr.astype(jnp.bfloat16).astype(jnp.float32)
    d_lo = dr - d_mid_f
    row8 = jax.lax.broadcasted_iota(jnp.int32, (8, BN), 0)
    m0 = row8 == 0
    xc_ref[STEPS:SP, :] = jnp.where(m0, drift_dt, 0.0).astype(jnp.bfloat16)
    xc_ref[SP + STEPS:2 * SP, :] = jnp.where(m0, dr, 0.0).astype(jnp.bfloat16)
    xc_ref[2 * SP + STEPS:3 * SP, :] = jnp.where(m0, d_lo, 0.0).astype(jnp.bfloat16)

    s = jax.lax.dot(w_ref[...], xc_ref[...],
                    preferred_element_type=jnp.float32)

    t_f = jax.lax.broadcasted_iota(
        jnp.int32, (STEPS, BN), 0).astype(jnp.float32)
    enc_u = jnp.where(s >= th_hi, t_f, BIGF)
    enc_l = jnp.where(s <= th_lo, t_f, BIGF)
    mu = jnp.min(enc_u, axis=0, keepdims=True)
    ml = jnp.min(enc_l, axis=0, keepdims=True)

    t_first = jnp.minimum(mu, ml)
    hit = t_first < BIGF
    pr_ref[...] = jnp.where(hit, t_first * DT + ndt, MAX_T + ndt)
    pc_ref[...] = jnp.where(hit, jnp.where(mu <= ml, 1.0, 0.0), 0.5)


def _weights():
    tril = np.tril(np.ones((STEPS, STEPS), np.float32))
    tcol = np.arange(1, STEPS + 1, dtype=np.float32).reshape(STEPS, 1)
    sec = np.concatenate([tril, tcol, np.zeros((STEPS, SP - STEPS - 1),
                                               np.float32)], axis=1)
    return jnp.asarray(np.concatenate([sec] * 3, axis=1), dtype=jnp.bfloat16)


@jax.jit
def kernel(x, a, z, ndt, drift_gain, noise):
    n = x.shape[0]
    x2 = x.reshape(1, n)
    w = _weights()
    grid = (pl.cdiv(n, BN),)
    scal = pl.BlockSpec(memory_space=pltpu.SMEM)
    pr, pc = pl.pallas_call(
        _ddm_block,
        grid=grid,
        in_specs=[
            pl.BlockSpec((1, BN), lambda i: (0, i)),
            pl.BlockSpec((STEPS, 3 * SP), lambda i: (0, 0)),
            scal, scal, scal, scal,
            pl.BlockSpec((STEPS, BN), lambda i: (0, i)),
        ],
        out_specs=[
            pl.BlockSpec((1, BN), lambda i: (0, i)),
            pl.BlockSpec((1, BN), lambda i: (0, i)),
        ],
        out_shape=[
            jax.ShapeDtypeStruct((1, n), jnp.float32),
            jax.ShapeDtypeStruct((1, n), jnp.float32),
        ],
        scratch_shapes=[pltpu.VMEM((3 * SP, BN), jnp.bfloat16)],
        compiler_params=pltpu.CompilerParams(
            dimension_semantics=("parallel",)),
    )(x2, w,
      a.reshape(1, 1), z.reshape(1, 1), ndt.reshape(1, 1),
      drift_gain.reshape(1, 1), noise)
    return pr.reshape(n), pc.reshape(n)


# speedup vs baseline: 1.0020x; 1.0002x over previous
import numpy as np
import jax
import jax.numpy as jnp
from jax.experimental import pallas as pl
from jax.experimental.pallas import tpu as pltpu

DT = 0.01
MAX_T = 2.0
STEPS = 200
SP = 208
BN = 5120
BIGF = 1e9


def _ddm_block(x_ref, w_ref, a_ref, z_ref, ndt_ref, g_ref, noise_ref,
               pr_ref, pc_ref, xc_ref):
    a = a_ref[0, 0]
    z = z_ref[0, 0]
    ndt = ndt_ref[0, 0]
    gain = g_ref[0, 0]

    drift_dt = (gain * DT) * x_ref[...]
    th_hi = (a - 1e-6) - z * a
    th_lo = (-a + 1e-6) - z * a

    nz = noise_ref[...]
    hi = nz.astype(jnp.bfloat16)
    r1 = nz - hi.astype(jnp.float32)
    mid = r1.astype(jnp.bfloat16)
    lo = (r1 - mid.astype(jnp.float32)).astype(jnp.bfloat16)
    xc_ref[0:STEPS, :] = hi
    xc_ref[SP:SP + STEPS, :] = mid
    xc_ref[2 * SP:2 * SP + STEPS, :] = lo

    d_hi_f = drift_dt.astype(jnp.bfloat16).astype(jnp.float32)
    dr = drift_dt - d_hi_f
    d_mid_f = dr.astype(jnp.bfloat16).astype(jnp.float32)
    d_lo = dr - d_mid_f
    row8 = jax.lax.broadcasted_iota(jnp.int32, (8, BN), 0)
    m0 = row8 == 0
    xc_ref[STEPS:SP, :] = jnp.where(m0, drift_dt, 0.0).astype(jnp.bfloat16)
    xc_ref[SP + STEPS:2 * SP, :] = jnp.where(m0, dr, 0.0).astype(jnp.bfloat16)
    xc_ref[2 * SP + STEPS:3 * SP, :] = jnp.where(m0, d_lo, 0.0).astype(jnp.bfloat16)

    s = jax.lax.dot(w_ref[...], xc_ref[...],
                    preferred_element_type=jnp.float32)

    t_f = jax.lax.broadcasted_iota(
        jnp.int32, (STEPS, BN), 0).astype(jnp.float32)
    enc_u = jnp.where(s >= th_hi, t_f, BIGF)
    enc_l = jnp.where(s <= th_lo, t_f, BIGF)
    mu = jnp.min(enc_u, axis=0, keepdims=True)
    ml = jnp.min(enc_l, axis=0, keepdims=True)

    t_first = jnp.minimum(mu, ml)
    hit = t_first < BIGF
    pr_ref[...] = jnp.where(hit, t_first * DT + ndt, MAX_T + ndt)
    pc_ref[...] = jnp.where(hit, jnp.where(mu <= ml, 1.0, 0.0), 0.5)


def _weights():
    tril = np.tril(np.ones((STEPS, STEPS), np.float32))
    tcol = np.arange(1, STEPS + 1, dtype=np.float32).reshape(STEPS, 1)
    sec = np.concatenate([tril, tcol, np.zeros((STEPS, SP - STEPS - 1),
                                               np.float32)], axis=1)
    return jnp.asarray(np.concatenate([sec] * 3, axis=1), dtype=jnp.bfloat16)


@jax.jit
def kernel(x, a, z, ndt, drift_gain, noise):
    n = x.shape[0]
    x2 = x.reshape(1, n)
    w = _weights()
    grid = (pl.cdiv(n, BN),)
    scal = pl.BlockSpec(memory_space=pltpu.SMEM)
    pr, pc = pl.pallas_call(
        _ddm_block,
        grid=grid,
        in_specs=[
            pl.BlockSpec((1, BN), lambda i: (0, i)),
            pl.BlockSpec((STEPS, 3 * SP), lambda i: (0, 0)),
            scal, scal, scal, scal,
            pl.BlockSpec((STEPS, BN), lambda i: (0, i)),
        ],
        out_specs=[
            pl.BlockSpec((1, BN), lambda i: (0, i)),
            pl.BlockSpec((1, BN), lambda i: (0, i)),
        ],
        out_shape=[
            jax.ShapeDtypeStruct((1, n), jnp.float32),
            jax.ShapeDtypeStruct((1, n), jnp.float32),
        ],
        scratch_shapes=[pltpu.VMEM((3 * SP, BN), jnp.bfloat16)],
    )(x2, w,
      a.reshape(1, 1), z.reshape(1, 1), ndt.reshape(1, 1),
      drift_gain.reshape(1, 1), noise)
    return pr.reshape(n), pc.reshape(n)
